# baseline (speedup 1.0000x reference)
import functools

import jax
import jax.numpy as jnp
from jax import lax
from jax.experimental import pallas as pl
from jax.experimental.pallas import tpu as pltpu
from jax.experimental.pallas import tpu_sc as plsc

_N = 50000
_E = 800000
_B = 30
_T = 5
_H = 100
_NT = 16
_EB = 1024
_EEB = 7168
_EPT = 7
_EP = _NT * _EPT * _EEB
_NP = 51200
_NBN = _NP // _EB
_GW = 160
_TRASH = 158
_NCH = 8
_CR = _NP // _NCH
_CW = _CR * _GW
_CPAD = _CW + 8
_BP = 32


def _rsqrt16(d):
    i = lax.bitcast_convert_type(d, jnp.int32)
    i = jnp.int32(0x5F3759DF) - lax.shift_right_logical(i, 1)
    y = lax.bitcast_convert_type(i, jnp.float32)
    for _ in range(3):
        y = y * (1.5 - 0.5 * d * y * y)
    return y


def _sc_body(x_h, src_h, dst_h, batch_h, a1_o, cnt_o, c_o, w_o, ge_o,
             deg_s, s1_s, dis_s, col_s, cnt_s, cc_s,
             zbuf, obuf, onb, ia, ib, ic, fa,
             na, nb, nc, nd, nia, nib, i4, f4):
    c = lax.axis_index("c")
    s = lax.axis_index("s")
    nblk_node = (_NBN + _NT - 1) // _NT

    def fill(i, _):
        zbuf[pl.ds(i * 16, 16)] = jnp.zeros((16,), jnp.float32)
        obuf[pl.ds(i * 16, 16)] = jnp.ones((16,), jnp.float32)
        return 0
    lax.fori_loop(0, _EEB // 16, fill, 0)
    def fill2(i, _):
        onb[pl.ds(i * 16, 16)] = jnp.ones((16,), jnp.float32)
        return 0
    lax.fori_loop(0, _EB // 16, fill2, 0)

    def p0_blk(b, _):
        blk = s + _NT * b
        @pl.when(blk < _NBN)
        def _():
            off = pl.multiple_of(blk * _EB, _EB)
            pltpu.sync_copy(zbuf.at[pl.ds(0, _EB)], deg_s.at[pl.ds(off, _EB)])
            pltpu.sync_copy(zbuf.at[pl.ds(0, _EB)], s1_s.at[pl.ds(off, _EB)])
            pltpu.sync_copy(batch_h.at[pl.ds(off, _EB)], nia)
            pltpu.sync_copy(nia, col_s.at[pl.ds(off, _EB)])
        return 0
    lax.fori_loop(0, nblk_node, p0_blk, 0)
    @pl.when(s == 0)
    def _():
        pltpu.sync_copy(zbuf.at[pl.ds(0, 256)], cnt_s)
    plsc.subcore_barrier()

    def deg_blk(b, _):
        eoff = pl.multiple_of((s * _EPT + b) * _EEB, _EEB)
        pltpu.sync_copy(dst_h.at[pl.ds(eoff, _EEB)], ib)
        pltpu.sync_copy(obuf, deg_s.at[ib], add=True)
        return 0
    with jax.named_scope("P1_deg"):
        lax.fori_loop(0, _EPT, deg_blk, 0)
    plsc.subcore_barrier()

    def ew1_blk(b, _):
        blk = s + _NT * b
        @pl.when(blk < _NBN)
        def _():
            off = pl.multiple_of(blk * _EB, _EB)
            pltpu.sync_copy(deg_s.at[pl.ds(off, _EB)], na)
            pltpu.sync_copy(x_h.at[pl.ds(off, _EB)], nb)
            def inner(j, _):
                sl = pl.ds(j * 16, 16)
                r = _rsqrt16(na[sl] + 1.0)
                nc[sl] = r
                nd[sl] = r * nb[sl]
                return 0
            lax.fori_loop(0, _EB // 16, inner, 0)
            pltpu.sync_copy(nc, dis_s.at[pl.ds(off, _EB)])
            pltpu.sync_copy(nd, deg_s.at[pl.ds(off, _EB)])
        return 0
    with jax.named_scope("P2_dis"):
        lax.fori_loop(0, nblk_node, ew1_blk, 0)
    plsc.subcore_barrier()

    def p3_blk(b, _):
        eoff = pl.multiple_of((s * _EPT + b) * _EEB, _EEB)
        pltpu.sync_copy(src_h.at[pl.ds(eoff, _EEB)], ia)
        pltpu.sync_copy(dst_h.at[pl.ds(eoff, _EEB)], ib)
        pltpu.sync_copy(deg_s.at[ia], fa)
        pltpu.sync_copy(fa, s1_s.at[ib], add=True)
        pltpu.sync_copy(dis_s.at[ia], fa)
        pltpu.sync_copy(dis_s.at[ib], obuf)
        pltpu.sync_copy(col_s.at[ib], ic)
        def inner(j, _):
            sl = pl.ds(j * 16, 16)
            obuf[sl] = fa[sl] * obuf[sl]
            g = ic[sl]
            q = lax.div(g, jnp.int32(5))
            r = g - q * 5
            ic[sl] = jnp.where(g < 150, r * _BP + q, jnp.int32(_TRASH))
            return 0
        lax.fori_loop(0, _EEB // 16, inner, 0)
        pltpu.sync_copy(obuf, w_o.at[pl.ds(eoff, _EEB)])
        pltpu.sync_copy(ic, ge_o.at[pl.ds(eoff, _EEB)])
        return 0
    with jax.named_scope("P3_s1_w"):
        lax.fori_loop(0, _EPT, p3_blk, 0)
    plsc.subcore_barrier()

    def p4_blk(b, _):
        blk = s + _NT * b
        @pl.when(blk < _NBN)
        def _():
            off = pl.multiple_of(blk * _EB, _EB)
            pltpu.sync_copy(dis_s.at[pl.ds(off, _EB)], na)
            pltpu.sync_copy(s1_s.at[pl.ds(off, _EB)], nb)
            pltpu.sync_copy(deg_s.at[pl.ds(off, _EB)], nc)
            pltpu.sync_copy(col_s.at[pl.ds(off, _EB)], nia)
            def inner(j, _):
                sl = pl.ds(j * 16, 16)
                nd[sl] = na[sl] * (nb[sl] + nc[sl])
                g = nia[sl]
                q = lax.div(g, jnp.int32(5))
                r = g - q * 5
                nib[sl] = jnp.where(g < 150, r * _BP + q, jnp.int32(_TRASH))
                return 0
            lax.fori_loop(0, _EB // 16, inner, 0)
            pltpu.sync_copy(nib, col_s.at[pl.ds(off, _EB)])
            @pl.when(c == 0)
            def _():
                pltpu.sync_copy(nd, a1_o.at[pl.ds(off, _EB)])
                pltpu.sync_copy(onb, cnt_s.at[nib], add=True)
        return 0
    with jax.named_scope("P4_a1"):
        lax.fori_loop(0, nblk_node, p4_blk, 0)
    plsc.subcore_barrier()
    @pl.when(jnp.logical_and(c == 0, s == 0))
    def _():
        pltpu.sync_copy(cnt_s, cnt_o)

    nzf = _CW // _EEB
    nzr = _CW - nzf * _EEB
    for k in range(_NCH // 2):
        cid = c * (_NCH // 2) + k
        base = cid * _CR
        if True:
            def z_blk(b, _):
                blk = s + _NT * b
                @pl.when(blk < nzf)
                def _():
                    pltpu.sync_copy(zbuf, cc_s.at[pl.ds(blk * _EEB, _EEB)])
                return 0
            with jax.named_scope("P5_zero"):
                lax.fori_loop(0, (nzf + _NT - 1) // _NT, z_blk, 0)
                @pl.when(s == 0)
                def _():
                    pltpu.sync_copy(zbuf.at[pl.ds(0, nzr)],
                                    cc_s.at[pl.ds(nzf * _EEB, nzr)])
            plsc.subcore_barrier()
            def ce_blk(b, _):
                eoff = pl.multiple_of((s * _EPT + b) * _EEB, _EEB)
                pltpu.sync_copy(src_h.at[pl.ds(eoff, _EEB)], ia)
                pltpu.sync_copy(w_o.at[pl.ds(eoff, _EEB)], fa)
                pltpu.sync_copy(ge_o.at[pl.ds(eoff, _EEB)], ic)
                def inner(j, _):
                    sl = pl.ds(j * 16, 16)
                    sv = ia[sl]
                    loc = sv - base
                    ok = jnp.logical_and(sv >= base, loc < _CR)
                    ib[sl] = jnp.where(ok, loc * _GW + ic[sl], jnp.int32(_CW))
                    return 0
                lax.fori_loop(0, _EEB // 16, inner, 0)
                pltpu.sync_copy(fa, cc_s.at[ib], add=True)
                return 0
            with jax.named_scope("P5_scatter"):
                lax.fori_loop(0, _EPT, ce_blk, 0)
            npt = _CR // _NT
            soff = pl.multiple_of(base + s * npt, 8)
            pltpu.sync_copy(dis_s.at[pl.ds(soff, npt)], f4)
            pltpu.sync_copy(col_s.at[pl.ds(soff, npt)], i4)
            def self_blk(j, _):
                sl = pl.ds(j * 16, 16)
                f4[sl] = f4[sl] * f4[sl]
                loc = s * npt + j * 16 + lax.iota(jnp.int32, 16)
                i4[sl] = loc * _GW + i4[sl]
                return 0
            lax.fori_loop(0, npt // 16, self_blk, 0)
            pltpu.sync_copy(f4, cc_s.at[i4], add=True)
            plsc.subcore_barrier()
            with jax.named_scope("P5_flush"):
                tw = _CW // _NT
                pltpu.sync_copy(cc_s.at[pl.ds(s * tw, tw)],
                                c_o.at[pl.ds(cid * _CW + s * tw, tw)])
            plsc.subcore_barrier()


def _make_sc_kernel():
    return functools.partial(
        pl.kernel,
        out_type=[
            jax.ShapeDtypeStruct((_NP,), jnp.float32),
            jax.ShapeDtypeStruct((256,), jnp.float32),
            jax.ShapeDtypeStruct((_NCH * _CW,), jnp.float32),
            jax.ShapeDtypeStruct((_EP,), jnp.float32),
            jax.ShapeDtypeStruct((_EP,), jnp.int32),
        ],
        mesh=plsc.VectorSubcoreMesh(core_axis_name="c", subcore_axis_name="s"),
        scratch_types=[
            pltpu.VMEM_SHARED((_NP,), jnp.float32),
            pltpu.VMEM_SHARED((_NP,), jnp.float32),
            pltpu.VMEM_SHARED((_NP,), jnp.float32),
            pltpu.VMEM_SHARED((_NP,), jnp.int32),
            pltpu.VMEM_SHARED((256,), jnp.float32),
            pltpu.VMEM_SHARED((_CPAD,), jnp.float32),
            pltpu.VMEM((_EEB,), jnp.float32),
            pltpu.VMEM((_EEB,), jnp.float32),
            pltpu.VMEM((_EB,), jnp.float32),
            pltpu.VMEM((_EEB,), jnp.int32),
            pltpu.VMEM((_EEB,), jnp.int32),
            pltpu.VMEM((_EEB,), jnp.int32),
            pltpu.VMEM((_EEB,), jnp.float32),
            pltpu.VMEM((_EB,), jnp.float32),
            pltpu.VMEM((_EB,), jnp.float32),
            pltpu.VMEM((_EB,), jnp.float32),
            pltpu.VMEM((_EB,), jnp.float32),
            pltpu.VMEM((_EB,), jnp.int32),
            pltpu.VMEM((_EB,), jnp.int32),
            pltpu.VMEM((_CR // _NT,), jnp.int32),
            pltpu.VMEM((_CR // _NT,), jnp.float32),
        ],
    )(_sc_body)


def _tc_body(a1_ref, c_ref, cnt_ref, W1_ref, W2_ref, b1_ref, b2_ref,
             Wih_ref, Whh_ref, bih_ref, bhh_ref, Wout_ref, bout_ref,
             out_ref, acc_ref):
    i = pl.program_id(0)
    @pl.when(i == 0)
    def _():
        acc_ref[...] = jnp.zeros((_GW, _H), jnp.float32)
    bf = jnp.bfloat16
    f32 = jnp.float32
    hi = lax.Precision.HIGHEST
    out1 = a1_ref[...] * W1_ref[...] + b1_ref[...]
    h2 = lax.dot_general(out1.astype(bf), W2_ref[...].astype(bf),
                         (((1,), (0,)), ((), ())), preferred_element_type=f32)
    acc_ref[...] += lax.dot_general(c_ref[...], h2, (((0,), (0,)), ((), ())),
                                    precision=hi, preferred_element_type=f32)
    @pl.when(i == pl.num_programs(0) - 1)
    def _():
        cnt = jnp.maximum(cnt_ref[...][0, :_GW], 1.0)
        pooled = acc_ref[...] / cnt[:, None] + b2_ref[...]
        Wih = Wih_ref[...].astype(bf)
        Whh = Whh_ref[...].astype(bf)
        bias = bih_ref[...] + bhh_ref[...]
        dn = (((1,), (1,)), ((), ()))
        h = jnp.zeros((_BP, _H), f32)
        cc = jnp.zeros((_BP, _H), f32)
        for t in range(_T):
            xt = pooled[t * _BP:(t + 1) * _BP]
            gates = (lax.dot_general(xt.astype(bf), Wih, dn,
                                     preferred_element_type=f32)
                     + lax.dot_general(h.astype(bf), Whh, dn,
                                       preferred_element_type=f32) + bias)
            ig = jax.nn.sigmoid(gates[:, :_H])
            fg = jax.nn.sigmoid(gates[:, _H:2 * _H])
            gg = jnp.tanh(gates[:, 2 * _H:3 * _H])
            og = jax.nn.sigmoid(gates[:, 3 * _H:])
            cc = fg * cc + ig * gg
            h = og * jnp.tanh(cc)
        out_ref[...] = (lax.dot_general(h.astype(bf), Wout_ref[...].astype(bf),
                                        (((1,), (0,)), ((), ())),
                                        preferred_element_type=f32)
                        + bout_ref[...])


@jax.jit
def kernel(x, edge_index, batch, W1, b1, W2, b2, W_ih, W_hh, b_ih, b_hh,
           W_out, b_out):
    xp = jnp.pad(x[:, 0], (0, _NP - _N))
    pad_idx = (_N + jnp.arange(_EP - _E, dtype=jnp.int32) % 1024)
    srcp = jnp.concatenate([edge_index[0], pad_idx])
    dstp = jnp.concatenate([edge_index[1], pad_idx])
    batchp = jnp.pad(batch, (0, _NP - _N), constant_values=200)

    a1, cnt, cflat, _, _ = _make_sc_kernel()(xp, srcp, dstp, batchp)
    C = cflat.reshape(_NP, _GW)

    nblk = 8
    blk = _NP // nblk
    pred = pl.pallas_call(
        _tc_body,
        grid=(nblk,),
        in_specs=[
            pl.BlockSpec((blk, 1), lambda i: (i, 0)),
            pl.BlockSpec((blk, _GW), lambda i: (i, 0)),
            pl.BlockSpec((1, 256), lambda i: (0, 0)),
            pl.BlockSpec((1, _H), lambda i: (0, 0)),
            pl.BlockSpec((_H, _H), lambda i: (0, 0)),
            pl.BlockSpec((1, _H), lambda i: (0, 0)),
            pl.BlockSpec((1, _H), lambda i: (0, 0)),
            pl.BlockSpec((4 * _H, _H), lambda i: (0, 0)),
            pl.BlockSpec((4 * _H, _H), lambda i: (0, 0)),
            pl.BlockSpec((1, 4 * _H), lambda i: (0, 0)),
            pl.BlockSpec((1, 4 * _H), lambda i: (0, 0)),
            pl.BlockSpec((_H, 1), lambda i: (0, 0)),
            pl.BlockSpec((1, 1), lambda i: (0, 0)),
        ],
        out_specs=pl.BlockSpec((_BP, 1), lambda i: (0, 0)),
        out_shape=jax.ShapeDtypeStruct((_BP, 1), jnp.float32),
        scratch_shapes=[pltpu.VMEM((_GW, _H), jnp.float32)],
    )(a1.reshape(_NP, 1), C, cnt.reshape(1, 256), W1, W2,
      b1.reshape(1, _H), b2.reshape(1, _H), W_ih, W_hh,
      b_ih.reshape(1, 4 * _H), b_hh.reshape(1, 4 * _H),
      W_out, b_out.reshape(1, 1))
    return pred[:_B]


# speedup vs baseline: 5.2583x; 5.2583x over previous
import functools

import jax
import jax.numpy as jnp
from jax import lax
from jax.experimental import pallas as pl
from jax.experimental.pallas import tpu as pltpu
from jax.experimental.pallas import tpu_sc as plsc

_N = 50000
_E = 800000
_B = 30
_T = 5
_H = 100
_NT = 16
_EB = 1024
_EEB = 7168
_EPT = 7
_EP = _NT * _EPT * _EEB
_NP = 51200
_NBN = _NP // _EB
_GW = 160
_TRASH = 158
_NCH = 8
_CR = _NP // _NCH
_CW = _CR * _GW
_CPAD = _CW + 1024
_BP = 32


def _rsqrt16(d):
    i = lax.bitcast_convert_type(d, jnp.int32)
    i = jnp.int32(0x5F3759DF) - lax.shift_right_logical(i, 1)
    y = lax.bitcast_convert_type(i, jnp.float32)
    for _ in range(3):
        y = y * (1.5 - 0.5 * d * y * y)
    return y


def _sc_body(x_h, src_h, dst_h, batch_h, a1_o, cnt_o, c_o, w_o, ge_o,
             deg_s, s1_s, dis_s, col_s, cnt_s, cc_s,
             zbuf, obuf, onb, ia, ib, ic, fa,
             na, nb, nc, nd, nia, nib, i4, f4):
    c = lax.axis_index("c")
    s = lax.axis_index("s")
    nblk_node = (_NBN + _NT - 1) // _NT

    def fill(i, _):
        zbuf[pl.ds(i * 16, 16)] = jnp.zeros((16,), jnp.float32)
        obuf[pl.ds(i * 16, 16)] = jnp.ones((16,), jnp.float32)
        return 0
    lax.fori_loop(0, _EEB // 16, fill, 0)
    def fill2(i, _):
        onb[pl.ds(i * 16, 16)] = jnp.ones((16,), jnp.float32)
        return 0
    lax.fori_loop(0, _EB // 16, fill2, 0)

    def p0_blk(b, _):
        blk = s + _NT * b
        @pl.when(blk < _NBN)
        def _():
            off = pl.multiple_of(blk * _EB, _EB)
            pltpu.sync_copy(zbuf.at[pl.ds(0, _EB)], deg_s.at[pl.ds(off, _EB)])
            pltpu.sync_copy(zbuf.at[pl.ds(0, _EB)], s1_s.at[pl.ds(off, _EB)])
            pltpu.sync_copy(batch_h.at[pl.ds(off, _EB)], nia)
            pltpu.sync_copy(nia, col_s.at[pl.ds(off, _EB)])
        return 0
    lax.fori_loop(0, nblk_node, p0_blk, 0)
    @pl.when(s == 0)
    def _():
        pltpu.sync_copy(zbuf.at[pl.ds(0, 256)], cnt_s)
    plsc.subcore_barrier()

    def deg_blk(b, _):
        eoff = pl.multiple_of((s * _EPT + b) * _EEB, _EEB)
        pltpu.sync_copy(dst_h.at[pl.ds(eoff, _EEB)], ib)
        pltpu.sync_copy(obuf, deg_s.at[ib], add=True)
        return 0
    with jax.named_scope("P1_deg"):
        lax.fori_loop(0, _EPT, deg_blk, 0)
    plsc.subcore_barrier()

    def ew1_blk(b, _):
        blk = s + _NT * b
        @pl.when(blk < _NBN)
        def _():
            off = pl.multiple_of(blk * _EB, _EB)
            pltpu.sync_copy(deg_s.at[pl.ds(off, _EB)], na)
            pltpu.sync_copy(x_h.at[pl.ds(off, _EB)], nb)
            def inner(j, _):
                sl = pl.ds(j * 16, 16)
                r = _rsqrt16(na[sl] + 1.0)
                nc[sl] = r
                nd[sl] = r * nb[sl]
                return 0
            lax.fori_loop(0, _EB // 16, inner, 0)
            pltpu.sync_copy(nc, dis_s.at[pl.ds(off, _EB)])
            pltpu.sync_copy(nd, deg_s.at[pl.ds(off, _EB)])
        return 0
    with jax.named_scope("P2_dis"):
        lax.fori_loop(0, nblk_node, ew1_blk, 0)
    plsc.subcore_barrier()

    def p3_blk(b, _):
        eoff = pl.multiple_of((s * _EPT + b) * _EEB, _EEB)
        pltpu.sync_copy(src_h.at[pl.ds(eoff, _EEB)], ia)
        pltpu.sync_copy(dst_h.at[pl.ds(eoff, _EEB)], ib)
        pltpu.sync_copy(deg_s.at[ia], fa)
        pltpu.sync_copy(fa, s1_s.at[ib], add=True)
        pltpu.sync_copy(dis_s.at[ia], fa)
        pltpu.sync_copy(dis_s.at[ib], obuf)
        pltpu.sync_copy(col_s.at[ib], ic)
        def inner(j, _):
            sl = pl.ds(j * 16, 16)
            obuf[sl] = fa[sl] * obuf[sl]
            g = ic[sl]
            q = lax.div(g, jnp.int32(5))
            r = g - q * 5
            ic[sl] = jnp.where(g < 150, r * _BP + q, jnp.int32(_TRASH))
            return 0
        lax.fori_loop(0, _EEB // 16, inner, 0)
        pltpu.sync_copy(obuf, w_o.at[pl.ds(eoff, _EEB)])
        pltpu.sync_copy(ic, ge_o.at[pl.ds(eoff, _EEB)])
        return 0
    with jax.named_scope("P3_s1_w"):
        lax.fori_loop(0, _EPT, p3_blk, 0)
    plsc.subcore_barrier()

    def p4_blk(b, _):
        blk = s + _NT * b
        @pl.when(blk < _NBN)
        def _():
            off = pl.multiple_of(blk * _EB, _EB)
            pltpu.sync_copy(dis_s.at[pl.ds(off, _EB)], na)
            pltpu.sync_copy(s1_s.at[pl.ds(off, _EB)], nb)
            pltpu.sync_copy(deg_s.at[pl.ds(off, _EB)], nc)
            pltpu.sync_copy(col_s.at[pl.ds(off, _EB)], nia)
            def inner(j, _):
                sl = pl.ds(j * 16, 16)
                nd[sl] = na[sl] * (nb[sl] + nc[sl])
                g = nia[sl]
                q = lax.div(g, jnp.int32(5))
                r = g - q * 5
                nib[sl] = jnp.where(g < 150, r * _BP + q, jnp.int32(_TRASH))
                return 0
            lax.fori_loop(0, _EB // 16, inner, 0)
            pltpu.sync_copy(nib, col_s.at[pl.ds(off, _EB)])
            @pl.when(c == 0)
            def _():
                pltpu.sync_copy(nd, a1_o.at[pl.ds(off, _EB)])
                pltpu.sync_copy(onb, cnt_s.at[nib], add=True)
        return 0
    with jax.named_scope("P4_a1"):
        lax.fori_loop(0, nblk_node, p4_blk, 0)
    plsc.subcore_barrier()
    @pl.when(jnp.logical_and(c == 0, s == 0))
    def _():
        pltpu.sync_copy(cnt_s, cnt_o)

    nzf = _CW // _EEB
    nzr = _CW - nzf * _EEB
    for k in range(_NCH // 2):
        cid = c * (_NCH // 2) + k
        base = cid * _CR
        if True:
            def z_blk(b, _):
                blk = s + _NT * b
                @pl.when(blk < nzf)
                def _():
                    pltpu.sync_copy(zbuf, cc_s.at[pl.ds(blk * _EEB, _EEB)])
                return 0
            with jax.named_scope("P5_zero"):
                lax.fori_loop(0, (nzf + _NT - 1) // _NT, z_blk, 0)
                @pl.when(s == 0)
                def _():
                    pltpu.sync_copy(zbuf.at[pl.ds(0, nzr)],
                                    cc_s.at[pl.ds(nzf * _EEB, nzr)])
            plsc.subcore_barrier()
            def ce_blk(b, _):
                eoff = pl.multiple_of((s * _EPT + b) * _EEB, _EEB)
                pltpu.sync_copy(src_h.at[pl.ds(eoff, _EEB)], ia)
                pltpu.sync_copy(w_o.at[pl.ds(eoff, _EEB)], fa)
                pltpu.sync_copy(ge_o.at[pl.ds(eoff, _EEB)], ic)
                def inner(j, _):
                    sl = pl.ds(j * 16, 16)
                    sv = ia[sl]
                    loc = sv - base
                    ok = jnp.logical_and(sv >= base, loc < _CR)
                    trash = (_CW + (j * 16) % 1024) + lax.iota(jnp.int32, 16)
                    ib[sl] = jnp.where(ok, loc * _GW + ic[sl], trash)
                    return 0
                lax.fori_loop(0, _EEB // 16, inner, 0)
                pltpu.sync_copy(fa, cc_s.at[ib], add=True)
                return 0
            with jax.named_scope("P5_scatter"):
                lax.fori_loop(0, _EPT, ce_blk, 0)
            npt = _CR // _NT
            soff = pl.multiple_of(base + s * npt, 8)
            pltpu.sync_copy(dis_s.at[pl.ds(soff, npt)], f4)
            pltpu.sync_copy(col_s.at[pl.ds(soff, npt)], i4)
            def self_blk(j, _):
                sl = pl.ds(j * 16, 16)
                f4[sl] = f4[sl] * f4[sl]
                loc = s * npt + j * 16 + lax.iota(jnp.int32, 16)
                i4[sl] = loc * _GW + i4[sl]
                return 0
            lax.fori_loop(0, npt // 16, self_blk, 0)
            pltpu.sync_copy(f4, cc_s.at[i4], add=True)
            plsc.subcore_barrier()
            with jax.named_scope("P5_flush"):
                tw = _CW // _NT
                pltpu.sync_copy(cc_s.at[pl.ds(s * tw, tw)],
                                c_o.at[pl.ds(cid * _CW + s * tw, tw)])
            plsc.subcore_barrier()


def _make_sc_kernel():
    return functools.partial(
        pl.kernel,
        out_type=[
            jax.ShapeDtypeStruct((_NP,), jnp.float32),
            jax.ShapeDtypeStruct((256,), jnp.float32),
            jax.ShapeDtypeStruct((_NCH * _CW,), jnp.float32),
            jax.ShapeDtypeStruct((_EP,), jnp.float32),
            jax.ShapeDtypeStruct((_EP,), jnp.int32),
        ],
        mesh=plsc.VectorSubcoreMesh(core_axis_name="c", subcore_axis_name="s"),
        scratch_types=[
            pltpu.VMEM_SHARED((_NP,), jnp.float32),
            pltpu.VMEM_SHARED((_NP,), jnp.float32),
            pltpu.VMEM_SHARED((_NP,), jnp.float32),
            pltpu.VMEM_SHARED((_NP,), jnp.int32),
            pltpu.VMEM_SHARED((256,), jnp.float32),
            pltpu.VMEM_SHARED((_CPAD,), jnp.float32),
            pltpu.VMEM((_EEB,), jnp.float32),
            pltpu.VMEM((_EEB,), jnp.float32),
            pltpu.VMEM((_EB,), jnp.float32),
            pltpu.VMEM((_EEB,), jnp.int32),
            pltpu.VMEM((_EEB,), jnp.int32),
            pltpu.VMEM((_EEB,), jnp.int32),
            pltpu.VMEM((_EEB,), jnp.float32),
            pltpu.VMEM((_EB,), jnp.float32),
            pltpu.VMEM((_EB,), jnp.float32),
            pltpu.VMEM((_EB,), jnp.float32),
            pltpu.VMEM((_EB,), jnp.float32),
            pltpu.VMEM((_EB,), jnp.int32),
            pltpu.VMEM((_EB,), jnp.int32),
            pltpu.VMEM((_CR // _NT,), jnp.int32),
            pltpu.VMEM((_CR // _NT,), jnp.float32),
        ],
    )(_sc_body)


def _tc_body(a1_ref, c_ref, cnt_ref, W1_ref, W2_ref, b1_ref, b2_ref,
             Wih_ref, Whh_ref, bih_ref, bhh_ref, Wout_ref, bout_ref,
             out_ref, acc_ref):
    i = pl.program_id(0)
    @pl.when(i == 0)
    def _():
        acc_ref[...] = jnp.zeros((_GW, _H), jnp.float32)
    bf = jnp.bfloat16
    f32 = jnp.float32
    hi = lax.Precision.HIGHEST
    out1 = a1_ref[...] * W1_ref[...] + b1_ref[...]
    h2 = lax.dot_general(out1.astype(bf), W2_ref[...].astype(bf),
                         (((1,), (0,)), ((), ())), preferred_element_type=f32)
    acc_ref[...] += lax.dot_general(c_ref[...], h2, (((0,), (0,)), ((), ())),
                                    precision=hi, preferred_element_type=f32)
    @pl.when(i == pl.num_programs(0) - 1)
    def _():
        cnt = jnp.maximum(cnt_ref[...][0, :_GW], 1.0)
        pooled = acc_ref[...] / cnt[:, None] + b2_ref[...]
        Wih = Wih_ref[...].astype(bf)
        Whh = Whh_ref[...].astype(bf)
        bias = bih_ref[...] + bhh_ref[...]
        dn = (((1,), (1,)), ((), ()))
        h = jnp.zeros((_BP, _H), f32)
        cc = jnp.zeros((_BP, _H), f32)
        for t in range(_T):
            xt = pooled[t * _BP:(t + 1) * _BP]
            gates = (lax.dot_general(xt.astype(bf), Wih, dn,
                                     preferred_element_type=f32)
                     + lax.dot_general(h.astype(bf), Whh, dn,
                                       preferred_element_type=f32) + bias)
            ig = jax.nn.sigmoid(gates[:, :_H])
            fg = jax.nn.sigmoid(gates[:, _H:2 * _H])
            gg = jnp.tanh(gates[:, 2 * _H:3 * _H])
            og = jax.nn.sigmoid(gates[:, 3 * _H:])
            cc = fg * cc + ig * gg
            h = og * jnp.tanh(cc)
        out_ref[...] = (lax.dot_general(h.astype(bf), Wout_ref[...].astype(bf),
                                        (((1,), (0,)), ((), ())),
                                        preferred_element_type=f32)
                        + bout_ref[...])


@jax.jit
def kernel(x, edge_index, batch, W1, b1, W2, b2, W_ih, W_hh, b_ih, b_hh,
           W_out, b_out):
    xp = jnp.pad(x[:, 0], (0, _NP - _N))
    pad_idx = (_N + jnp.arange(_EP - _E, dtype=jnp.int32) % 1024)
    srcp = jnp.concatenate([edge_index[0], pad_idx])
    dstp = jnp.concatenate([edge_index[1], pad_idx])
    batchp = jnp.pad(batch, (0, _NP - _N), constant_values=200)

    a1, cnt, cflat, _, _ = _make_sc_kernel()(xp, srcp, dstp, batchp)
    C = cflat.reshape(_NP, _GW)

    nblk = 8
    blk = _NP // nblk
    pred = pl.pallas_call(
        _tc_body,
        grid=(nblk,),
        in_specs=[
            pl.BlockSpec((blk, 1), lambda i: (i, 0)),
            pl.BlockSpec((blk, _GW), lambda i: (i, 0)),
            pl.BlockSpec((1, 256), lambda i: (0, 0)),
            pl.BlockSpec((1, _H), lambda i: (0, 0)),
            pl.BlockSpec((_H, _H), lambda i: (0, 0)),
            pl.BlockSpec((1, _H), lambda i: (0, 0)),
            pl.BlockSpec((1, _H), lambda i: (0, 0)),
            pl.BlockSpec((4 * _H, _H), lambda i: (0, 0)),
            pl.BlockSpec((4 * _H, _H), lambda i: (0, 0)),
            pl.BlockSpec((1, 4 * _H), lambda i: (0, 0)),
            pl.BlockSpec((1, 4 * _H), lambda i: (0, 0)),
            pl.BlockSpec((_H, 1), lambda i: (0, 0)),
            pl.BlockSpec((1, 1), lambda i: (0, 0)),
        ],
        out_specs=pl.BlockSpec((_BP, 1), lambda i: (0, 0)),
        out_shape=jax.ShapeDtypeStruct((_BP, 1), jnp.float32),
        scratch_shapes=[pltpu.VMEM((_GW, _H), jnp.float32)],
    )(a1.reshape(_NP, 1), C, cnt.reshape(1, 256), W1, W2,
      b1.reshape(1, _H), b2.reshape(1, _H), W_ih, W_hh,
      b_ih.reshape(1, 4 * _H), b_hh.reshape(1, 4 * _H),
      W_out, b_out.reshape(1, 1))
    return pred[:_B]
